# baseline (speedup 1.0000x reference)
import functools

import jax
import jax.numpy as jnp
from jax import lax
from jax.experimental import pallas as pl
from jax.experimental.pallas import tpu as pltpu
from jax.experimental.pallas import tpu_sc as plsc

N = 10000
D = 128
E = 320000
NW = 32
NSUB = 16
BATCH = 128
STEPS = -(-E // (NW * BATCH))
E_PAD = NW * BATCH * STEPS
NPAD = N + 240
SHARE = NPAD // NSUB
MPAD = N + 112
MSHARE = MPAD // NSUB
DUMP = MPAD - N

_mesh = plsc.VectorSubcoreMesh(core_axis_name="c", subcore_axis_name="s")


@functools.partial(
    pl.kernel,
    out_type=jax.ShapeDtypeStruct((2, NPAD), jnp.float32),
    mesh=_mesh,
    compiler_params=pltpu.CompilerParams(needs_layout_passes=False),
    scratch_types=[
        pltpu.VMEM_SHARED((NSUB, NPAD), jnp.float32),
        pltpu.VMEM((STEPS, BATCH), jnp.int32),
        pltpu.VMEM((NPAD,), jnp.float32),
        pltpu.VMEM((NSUB, SHARE), jnp.float32),
        pltpu.VMEM((SHARE,), jnp.float32),
    ],
)
def _deg_kernel(dst_hbm, out_hbm, spm, slab, hist, mbuf, rbuf):
    c = lax.axis_index("c")
    s = lax.axis_index("s")
    w = c * NSUB + s
    pltpu.sync_copy(dst_hbm.at[w], slab)
    zvec = jnp.zeros((16,), jnp.float32)

    def zbody(i, carry):
        hist[pl.ds(i * 16, 16)] = zvec
        return carry

    lax.fori_loop(0, NPAD // 16, zbody, 0)

    def body(j, carry):
        for k in range(BATCH // 16):
            idx16 = slab[j, pl.ds(k * 16, 16)]
            cnt, last = plsc.scan_count(idx16)
            plsc.addupdate_scatter(hist, [idx16], cnt.astype(jnp.float32),
                                   mask=last)
        return carry

    lax.fori_loop(0, STEPS, body, 0)
    pltpu.sync_copy(hist, spm.at[s])
    plsc.subcore_barrier()
    pltpu.sync_copy(spm.at[:, pl.ds(s * SHARE, SHARE)], mbuf)

    def mbody(ci, carry):
        acc16 = zvec
        for r in range(NSUB):
            acc16 = acc16 + mbuf[r, pl.ds(ci * 16, 16)]
        rbuf[pl.ds(ci * 16, 16)] = acc16
        return carry

    lax.fori_loop(0, SHARE // 16, mbody, 0)
    pltpu.sync_copy(rbuf, out_hbm.at[c, pl.ds(s * SHARE, SHARE)])


@functools.partial(
    pl.kernel,
    out_type=jax.ShapeDtypeStruct((2, MPAD, D), jnp.float32),
    mesh=_mesh,
    scratch_types=[
        pltpu.VMEM_SHARED((MPAD, D), jnp.float32),
        pltpu.VMEM((2, BATCH), jnp.int32),
        pltpu.VMEM((4, BATCH), jnp.int32),
        pltpu.VMEM((3, BATCH, D), jnp.float32),
        pltpu.SemaphoreType.DMA,
        pltpu.SemaphoreType.DMA,
        pltpu.SemaphoreType.DMA,
    ],
)
def _msg_kernel(hp_hbm, src_hbm, dst_hbm, zeros_hbm, out_hbm,
                acc, sbuf, dbuf, gbuf, semg, sems, semi):
    c = lax.axis_index("c")
    s = lax.axis_index("s")
    w = c * NSUB + s
    pltpu.async_copy(src_hbm.at[w, 0], sbuf.at[0], semi)
    pltpu.async_copy(dst_hbm.at[w, 0], dbuf.at[0], semi)
    pltpu.async_copy(src_hbm.at[w, 1], sbuf.at[1], semi)
    pltpu.async_copy(dst_hbm.at[w, 1], dbuf.at[1], semi)
    pltpu.sync_copy(zeros_hbm.at[pl.ds(s * MSHARE, MSHARE)],
                    acc.at[pl.ds(s * MSHARE, MSHARE)])
    pltpu.make_async_copy(src_hbm.at[w, 0], sbuf.at[0], semi).wait()
    pltpu.make_async_copy(dst_hbm.at[w, 0], dbuf.at[0], semi).wait()
    plsc.subcore_barrier()
    pltpu.async_copy(hp_hbm.at[sbuf.at[0]], gbuf.at[0], semg)

    def body(j, carry):
        bg = lax.rem(j, 3)
        bs = lax.rem(j, 2)
        bd = lax.rem(j, 4)
        pltpu.make_async_copy(hp_hbm.at[sbuf.at[bs]], gbuf.at[bg],
                              semg).wait()
        pltpu.async_copy(gbuf.at[bg], acc.at[dbuf.at[bd]], sems, add=True)

        @pl.when(j + 1 < STEPS)
        def _():
            bs1 = lax.rem(j + 1, 2)
            bd1 = lax.rem(j + 1, 4)
            bg1 = lax.rem(j + 1, 3)
            pltpu.make_async_copy(src_hbm.at[w, j + 1], sbuf.at[bs1],
                                  semi).wait()
            pltpu.make_async_copy(dst_hbm.at[w, j + 1], dbuf.at[bd1],
                                  semi).wait()

            @pl.when(j >= 2)
            def _():
                pltpu.make_async_copy(gbuf.at[bg1],
                                      acc.at[dbuf.at[lax.rem(j - 2, 4)]],
                                      sems).wait()

            pltpu.async_copy(hp_hbm.at[sbuf.at[bs1]], gbuf.at[bg1], semg)

            @pl.when(j + 2 < STEPS)
            def _():
                bs2 = lax.rem(j + 2, 2)
                bd2 = lax.rem(j + 2, 4)
                pltpu.async_copy(src_hbm.at[w, j + 2], sbuf.at[bs2], semi)
                pltpu.async_copy(dst_hbm.at[w, j + 2], dbuf.at[bd2], semi)

        return carry

    lax.fori_loop(0, STEPS, body, 0)
    pltpu.make_async_copy(gbuf.at[0], acc.at[dbuf.at[0]], sems).wait()
    pltpu.make_async_copy(gbuf.at[0], acc.at[dbuf.at[0]], sems).wait()
    pltpu.make_async_copy(gbuf.at[0], acc.at[dbuf.at[0]], sems).wait()
    plsc.subcore_barrier()
    pltpu.sync_copy(acc.at[pl.ds(s * MSHARE, MSHARE)],
                    out_hbm.at[c, pl.ds(s * MSHARE, MSHARE)])


_R = 1000


def _dinv_of(d_ref):
    return lax.rsqrt(d_ref[...] + 1.0)


def _tc_pre_body(x_ref, w_ref, d_ref, o_ref):
    dinv = _dinv_of(d_ref)
    o_ref[...] = dinv * jnp.dot(x_ref[...], w_ref[...],
                                preferred_element_type=jnp.float32)


def _tc_mid_body(a0_ref, a1_ref, hp_ref, d_ref, w_ref, b_ref, o_ref):
    dinv = _dinv_of(d_ref)
    h = dinv * (a0_ref[...] + a1_ref[...] + hp_ref[...]) + b_ref[...]
    h = jnp.maximum(h, 0.0)
    o_ref[...] = dinv * jnp.dot(h, w_ref[...],
                                preferred_element_type=jnp.float32)


def _tc_post_body(a0_ref, a1_ref, hp_ref, d_ref, b_ref, o_ref):
    dinv = _dinv_of(d_ref)
    h = dinv * (a0_ref[...] + a1_ref[...] + hp_ref[...]) + b_ref[...]
    o_ref[...] = jnp.maximum(h, 0.0)


def _row_spec():
    return pl.BlockSpec((_R, D), lambda i: (i, 0))


def _deg_spec():
    return pl.BlockSpec((_R, 1), lambda i: (i, 0))


def _full_spec(shape):
    return pl.BlockSpec(shape, lambda i: tuple(0 for _ in shape))


def _tc_pre(x, w, d):
    return pl.pallas_call(
        _tc_pre_body,
        grid=(N // _R,),
        in_specs=[_row_spec(), _full_spec((D, D)), _deg_spec()],
        out_specs=_row_spec(),
        out_shape=jax.ShapeDtypeStruct((N, D), jnp.float32),
    )(x, w, d)


def _tc_mid(a0, a1, hp, d, w, b):
    return pl.pallas_call(
        _tc_mid_body,
        grid=(N // _R,),
        in_specs=[_row_spec(), _row_spec(), _row_spec(), _deg_spec(),
                  _full_spec((D, D)), _full_spec((1, D))],
        out_specs=_row_spec(),
        out_shape=jax.ShapeDtypeStruct((N, D), jnp.float32),
    )(a0, a1, hp, d, w, b)


def _tc_post(a0, a1, hp, d, b):
    return pl.pallas_call(
        _tc_post_body,
        grid=(N // _R,),
        in_specs=[_row_spec(), _row_spec(), _row_spec(), _deg_spec(),
                  _full_spec((1, D))],
        out_specs=_row_spec(),
        out_shape=jax.ShapeDtypeStruct((N, D), jnp.float32),
    )(a0, a1, hp, d, b)


def kernel(x, edge_index, W1, b1, W2, b2):
    src = edge_index[0].astype(jnp.int32)
    dst = edge_index[1].astype(jnp.int32)
    n_pad = E_PAD - E
    pad_idx = jnp.arange(n_pad, dtype=jnp.int32)
    pad_src = (pad_idx * 997) % N
    pad_dst = N + pad_idx % DUMP
    src_sl = jnp.concatenate([src, pad_src]).reshape(NW, STEPS, BATCH)
    dst_sl = jnp.concatenate([dst, pad_dst]).reshape(NW, STEPS, BATCH)

    zeros_big = jnp.zeros((MPAD, D), jnp.float32)

    deg = _deg_kernel(dst_sl)
    dcol = (deg[0, :N] + deg[1, :N])[:, None]

    b1r = b1.reshape(1, D)
    b2r = b2.reshape(1, D)

    hp1 = _tc_pre(x, W1, dcol)
    acc1 = _msg_kernel(hp1, src_sl, dst_sl, zeros_big)
    hp2 = _tc_mid(acc1[0, :N], acc1[1, :N], hp1, dcol, W2, b1r)
    acc2 = _msg_kernel(hp2, src_sl, dst_sl, zeros_big)
    out = _tc_post(acc2[0, :N], acc2[1, :N], hp2, dcol, b2r)
    return out


# speedup vs baseline: 1.0073x; 1.0073x over previous
import functools

import jax
import jax.numpy as jnp
from jax import lax
from jax.experimental import pallas as pl
from jax.experimental.pallas import tpu as pltpu
from jax.experimental.pallas import tpu_sc as plsc

N = 10000
D = 128
E = 320000
NW = 32
NSUB = 16
BATCH = 128
STEPS = -(-E // (NW * BATCH))
E_PAD = NW * BATCH * STEPS
DUMP = 240
NPAD = N + DUMP
SHARE = NPAD // NSUB

_mesh = plsc.VectorSubcoreMesh(core_axis_name="c", subcore_axis_name="s")


@functools.partial(
    pl.kernel,
    out_type=jax.ShapeDtypeStruct((2, NPAD), jnp.float32),
    mesh=_mesh,
    compiler_params=pltpu.CompilerParams(needs_layout_passes=False),
    scratch_types=[
        pltpu.VMEM_SHARED((NSUB, NPAD), jnp.float32),
        pltpu.VMEM((STEPS, BATCH), jnp.int32),
        pltpu.VMEM((NPAD,), jnp.float32),
        pltpu.VMEM((NSUB, SHARE), jnp.float32),
        pltpu.VMEM((SHARE,), jnp.float32),
    ],
)
def _deg_kernel(dst_hbm, out_hbm, spm, slab, hist, mbuf, rbuf):
    c = lax.axis_index("c")
    s = lax.axis_index("s")
    w = c * NSUB + s
    pltpu.sync_copy(dst_hbm.at[w], slab)
    zvec = jnp.zeros((16,), jnp.float32)

    def zbody(i, carry):
        hist[pl.ds(i * 16, 16)] = zvec
        return carry

    lax.fori_loop(0, NPAD // 16, zbody, 0)

    def body(j, carry):
        for k in range(BATCH // 16):
            idx16 = slab[j, pl.ds(k * 16, 16)]
            cnt, last = plsc.scan_count(idx16)
            plsc.addupdate_scatter(hist, [idx16], cnt.astype(jnp.float32),
                                   mask=last)
        return carry

    lax.fori_loop(0, STEPS, body, 0)
    pltpu.sync_copy(hist, spm.at[s])
    plsc.subcore_barrier()
    pltpu.sync_copy(spm.at[:, pl.ds(s * SHARE, SHARE)], mbuf)

    def mbody(ci, carry):
        acc16 = zvec
        for r in range(NSUB):
            acc16 = acc16 + mbuf[r, pl.ds(ci * 16, 16)]
        rbuf[pl.ds(ci * 16, 16)] = acc16
        return carry

    lax.fori_loop(0, SHARE // 16, mbody, 0)
    pltpu.sync_copy(rbuf, out_hbm.at[c, pl.ds(s * SHARE, SHARE)])


@functools.partial(
    pl.kernel,
    out_type=jax.ShapeDtypeStruct((2, NPAD, D), jnp.float32),
    mesh=_mesh,
    scratch_types=[
        pltpu.VMEM_SHARED((NPAD, D), jnp.float32),
        pltpu.VMEM((STEPS, BATCH), jnp.int32),
        pltpu.VMEM((2, BATCH), jnp.int32),
        pltpu.VMEM((2, BATCH, D), jnp.float32),
        pltpu.SemaphoreType.DMA,
        pltpu.SemaphoreType.DMA,
    ],
)
def _msg_kernel(hp_hbm, src_hbm, dst_hbm, zeros_hbm, out_hbm,
                acc, srcv, dbuf, gbuf, semg, semi):
    c = lax.axis_index("c")
    s = lax.axis_index("s")
    w = c * NSUB + s
    pltpu.sync_copy(src_hbm.at[w], srcv)
    pltpu.async_copy(hp_hbm.at[srcv.at[0]], gbuf.at[0], semg)
    pltpu.async_copy(dst_hbm.at[w, 0], dbuf.at[0], semi)
    pltpu.sync_copy(zeros_hbm.at[pl.ds(s * SHARE, SHARE)],
                    acc.at[pl.ds(s * SHARE, SHARE)])
    plsc.subcore_barrier()

    def body(j, carry):
        b = lax.rem(j, 2)
        pltpu.make_async_copy(hp_hbm.at[srcv.at[j]], gbuf.at[b], semg).wait()
        pltpu.make_async_copy(dst_hbm.at[w, j], dbuf.at[b], semi).wait()

        @pl.when(j + 1 < STEPS)
        def _():
            pltpu.async_copy(hp_hbm.at[srcv.at[j + 1]], gbuf.at[1 - b], semg)
            pltpu.async_copy(dst_hbm.at[w, j + 1], dbuf.at[1 - b], semi)

        pltpu.sync_copy(gbuf.at[b], acc.at[dbuf.at[b]], add=True)
        return carry

    lax.fori_loop(0, STEPS, body, 0)
    plsc.subcore_barrier()
    pltpu.sync_copy(acc.at[pl.ds(s * SHARE, SHARE)],
                    out_hbm.at[c, pl.ds(s * SHARE, SHARE)])


_R = 1000


def _dinv_of(d_ref):
    return lax.rsqrt(d_ref[...] + 1.0)


def _tc_pre_body(x_ref, w_ref, d_ref, o_ref):
    dinv = _dinv_of(d_ref)
    o_ref[...] = dinv * jnp.dot(x_ref[...], w_ref[...],
                                preferred_element_type=jnp.float32)


def _tc_mid_body(a0_ref, a1_ref, hp_ref, d_ref, w_ref, b_ref, o_ref):
    dinv = _dinv_of(d_ref)
    h = dinv * (a0_ref[...] + a1_ref[...] + hp_ref[...]) + b_ref[...]
    h = jnp.maximum(h, 0.0)
    o_ref[...] = dinv * jnp.dot(h, w_ref[...],
                                preferred_element_type=jnp.float32)


def _tc_post_body(a0_ref, a1_ref, hp_ref, d_ref, b_ref, o_ref):
    dinv = _dinv_of(d_ref)
    h = dinv * (a0_ref[...] + a1_ref[...] + hp_ref[...]) + b_ref[...]
    o_ref[...] = jnp.maximum(h, 0.0)


def _row_spec():
    return pl.BlockSpec((_R, D), lambda i: (i, 0))


def _deg_spec():
    return pl.BlockSpec((_R, 1), lambda i: (i, 0))


def _full_spec(shape):
    return pl.BlockSpec(shape, lambda i: tuple(0 for _ in shape))


def _tc_pre(x, w, d):
    return pl.pallas_call(
        _tc_pre_body,
        grid=(N // _R,),
        in_specs=[_row_spec(), _full_spec((D, D)), _deg_spec()],
        out_specs=_row_spec(),
        out_shape=jax.ShapeDtypeStruct((N, D), jnp.float32),
    )(x, w, d)


def _tc_mid(a0, a1, hp, d, w, b):
    return pl.pallas_call(
        _tc_mid_body,
        grid=(N // _R,),
        in_specs=[_row_spec(), _row_spec(), _row_spec(), _deg_spec(),
                  _full_spec((D, D)), _full_spec((1, D))],
        out_specs=_row_spec(),
        out_shape=jax.ShapeDtypeStruct((N, D), jnp.float32),
    )(a0, a1, hp, d, w, b)


def _tc_post(a0, a1, hp, d, b):
    return pl.pallas_call(
        _tc_post_body,
        grid=(N // _R,),
        in_specs=[_row_spec(), _row_spec(), _row_spec(), _deg_spec(),
                  _full_spec((1, D))],
        out_specs=_row_spec(),
        out_shape=jax.ShapeDtypeStruct((N, D), jnp.float32),
    )(a0, a1, hp, d, b)


def kernel(x, edge_index, W1, b1, W2, b2):
    src = edge_index[0].astype(jnp.int32)
    dst = edge_index[1].astype(jnp.int32)
    n_pad = E_PAD - E
    pad_idx = jnp.arange(n_pad, dtype=jnp.int32)
    pad_src = (pad_idx * 997) % N
    pad_dst = N + pad_idx % DUMP
    src_sl = jnp.concatenate([src, pad_src]).reshape(NW, STEPS, BATCH)
    dst_sl = jnp.concatenate([dst, pad_dst]).reshape(NW, STEPS, BATCH)

    zeros_big = jnp.zeros((NPAD, D), jnp.float32)

    deg = _deg_kernel(dst_sl)
    dcol = (deg[0, :N] + deg[1, :N])[:, None]

    b1r = b1.reshape(1, D)
    b2r = b2.reshape(1, D)

    hp1 = _tc_pre(x, W1, dcol)
    acc1 = _msg_kernel(hp1, src_sl, dst_sl, zeros_big)
    hp2 = _tc_mid(acc1[0, :N], acc1[1, :N], hp1, dcol, W2, b1r)
    acc2 = _msg_kernel(hp2, src_sl, dst_sl, zeros_big)
    out = _tc_post(acc2[0, :N], acc2[1, :N], hp2, dcol, b2r)
    return out


# speedup vs baseline: 1.0124x; 1.0051x over previous
import functools

import jax
import jax.numpy as jnp
from jax import lax
from jax.experimental import pallas as pl
from jax.experimental.pallas import tpu as pltpu
from jax.experimental.pallas import tpu_sc as plsc

N = 10000
D = 128
E = 320000
NW = 32
NSUB = 16
BATCH = 128
STEPS = -(-E // (NW * BATCH))
E_PAD = NW * BATCH * STEPS
DUMP = 240
NPAD = N + DUMP
SHARE = NPAD // NSUB

_mesh = plsc.VectorSubcoreMesh(core_axis_name="c", subcore_axis_name="s")


@functools.partial(
    pl.kernel,
    out_type=jax.ShapeDtypeStruct((2, NPAD), jnp.float32),
    mesh=_mesh,
    compiler_params=pltpu.CompilerParams(needs_layout_passes=False),
    scratch_types=[
        pltpu.VMEM_SHARED((NSUB, NPAD), jnp.float32),
        pltpu.VMEM((STEPS, BATCH), jnp.int32),
        pltpu.VMEM((NPAD,), jnp.float32),
        pltpu.VMEM((NSUB, SHARE), jnp.float32),
        pltpu.VMEM((SHARE,), jnp.float32),
    ],
)
def _deg_kernel(dst_hbm, out_hbm, spm, slab, hist, mbuf, rbuf):
    c = lax.axis_index("c")
    s = lax.axis_index("s")
    w = c * NSUB + s
    pltpu.sync_copy(dst_hbm.at[w], slab)
    zvec = jnp.zeros((16,), jnp.float32)

    def zbody(i, carry):
        hist[pl.ds(i * 16, 16)] = zvec
        return carry

    lax.fori_loop(0, NPAD // 16, zbody, 0)

    def body(j, carry):
        for k in range(BATCH // 16):
            idx16 = slab[j, pl.ds(k * 16, 16)]
            cnt, last = plsc.scan_count(idx16)
            plsc.addupdate_scatter(hist, [idx16], cnt.astype(jnp.float32),
                                   mask=last)
        return carry

    lax.fori_loop(0, STEPS, body, 0)
    pltpu.sync_copy(hist, spm.at[s])
    plsc.subcore_barrier()
    pltpu.sync_copy(spm.at[:, pl.ds(s * SHARE, SHARE)], mbuf)

    def mbody(ci, carry):
        acc16 = zvec
        for r in range(NSUB):
            acc16 = acc16 + mbuf[r, pl.ds(ci * 16, 16)]
        rbuf[pl.ds(ci * 16, 16)] = acc16
        return carry

    lax.fori_loop(0, SHARE // 16, mbody, 0)
    pltpu.sync_copy(rbuf, out_hbm.at[c, pl.ds(s * SHARE, SHARE)])


@functools.partial(
    pl.kernel,
    out_type=jax.ShapeDtypeStruct((2, NPAD, D), jnp.float32),
    mesh=_mesh,
    scratch_types=[
        pltpu.VMEM_SHARED((NPAD, D), jnp.float32),
        pltpu.VMEM((STEPS, BATCH), jnp.int32),
        pltpu.VMEM((2, BATCH), jnp.int32),
        pltpu.VMEM((2, BATCH, D), jnp.float32),
        pltpu.SemaphoreType.DMA,
        pltpu.SemaphoreType.DMA,
    ],
)
def _msg_kernel(hp_hbm, src_hbm, dst_hbm, zeros_hbm, out_hbm,
                acc, srcv, dbuf, gbuf, semg, semi):
    c = lax.axis_index("c")
    s = lax.axis_index("s")
    w = c * NSUB + s
    pltpu.sync_copy(src_hbm.at[w], srcv)
    pltpu.async_copy(hp_hbm.at[srcv.at[0]], gbuf.at[0], semg)
    pltpu.async_copy(dst_hbm.at[w, 0], dbuf.at[0], semi)
    pltpu.sync_copy(zeros_hbm.at[pl.ds(s * SHARE, SHARE)],
                    acc.at[pl.ds(s * SHARE, SHARE)])
    plsc.subcore_barrier()

    def body(j, carry):
        b = lax.rem(j, 2)
        pltpu.make_async_copy(hp_hbm.at[srcv.at[j]], gbuf.at[b], semg).wait()
        pltpu.make_async_copy(dst_hbm.at[w, j], dbuf.at[b], semi).wait()

        @pl.when(j + 1 < STEPS)
        def _():
            pltpu.async_copy(hp_hbm.at[srcv.at[j + 1]], gbuf.at[1 - b], semg)
            pltpu.async_copy(dst_hbm.at[w, j + 1], dbuf.at[1 - b], semi)

        pltpu.sync_copy(gbuf.at[b], acc.at[dbuf.at[b]], add=True)
        return carry

    lax.fori_loop(0, STEPS, body, 0)
    plsc.subcore_barrier()
    pltpu.sync_copy(acc.at[pl.ds(s * SHARE, SHARE)],
                    out_hbm.at[c, pl.ds(s * SHARE, SHARE)])


_R = 1000


def _dinv_of(d_ref):
    return lax.rsqrt(d_ref[...] + 1.0)


def _tc_mm_body(x_ref, w_ref, o_ref):
    o_ref[...] = jnp.dot(x_ref[...], w_ref[...],
                         preferred_element_type=jnp.float32)


def _tc_scale_body(h_ref, d_ref, o_ref):
    o_ref[...] = _dinv_of(d_ref) * h_ref[...]


def _tc_mid_body(a0_ref, a1_ref, hp_ref, d_ref, w_ref, b_ref, o_ref):
    dinv = _dinv_of(d_ref)
    h = dinv * (a0_ref[...] + a1_ref[...] + hp_ref[...]) + b_ref[...]
    h = jnp.maximum(h, 0.0)
    o_ref[...] = dinv * jnp.dot(h, w_ref[...],
                                preferred_element_type=jnp.float32)


def _tc_post_body(a0_ref, a1_ref, hp_ref, d_ref, b_ref, o_ref):
    dinv = _dinv_of(d_ref)
    h = dinv * (a0_ref[...] + a1_ref[...] + hp_ref[...]) + b_ref[...]
    o_ref[...] = jnp.maximum(h, 0.0)


def _row_spec():
    return pl.BlockSpec((_R, D), lambda i: (i, 0))


def _deg_spec():
    return pl.BlockSpec((_R, 1), lambda i: (i, 0))


def _full_spec(shape):
    return pl.BlockSpec(shape, lambda i: tuple(0 for _ in shape))


def _tc_mm(x, w):
    return pl.pallas_call(
        _tc_mm_body,
        grid=(N // _R,),
        in_specs=[_row_spec(), _full_spec((D, D))],
        out_specs=_row_spec(),
        out_shape=jax.ShapeDtypeStruct((N, D), jnp.float32),
    )(x, w)


def _tc_scale(h, d):
    return pl.pallas_call(
        _tc_scale_body,
        grid=(N // _R,),
        in_specs=[_row_spec(), _deg_spec()],
        out_specs=_row_spec(),
        out_shape=jax.ShapeDtypeStruct((N, D), jnp.float32),
    )(h, d)


def _tc_mid(a0, a1, hp, d, w, b):
    return pl.pallas_call(
        _tc_mid_body,
        grid=(N // _R,),
        in_specs=[_row_spec(), _row_spec(), _row_spec(), _deg_spec(),
                  _full_spec((D, D)), _full_spec((1, D))],
        out_specs=_row_spec(),
        out_shape=jax.ShapeDtypeStruct((N, D), jnp.float32),
    )(a0, a1, hp, d, w, b)


def _tc_post(a0, a1, hp, d, b):
    return pl.pallas_call(
        _tc_post_body,
        grid=(N // _R,),
        in_specs=[_row_spec(), _row_spec(), _row_spec(), _deg_spec(),
                  _full_spec((1, D))],
        out_specs=_row_spec(),
        out_shape=jax.ShapeDtypeStruct((N, D), jnp.float32),
    )(a0, a1, hp, d, b)


def kernel(x, edge_index, W1, b1, W2, b2):
    src = edge_index[0].astype(jnp.int32)
    dst = edge_index[1].astype(jnp.int32)
    n_pad = E_PAD - E
    pad_idx = jnp.arange(n_pad, dtype=jnp.int32)
    pad_src = (pad_idx * 997) % N
    pad_dst = N + pad_idx % DUMP
    src_sl = jnp.concatenate([src, pad_src]).reshape(NW, STEPS, BATCH)
    dst_sl = jnp.concatenate([dst, pad_dst]).reshape(NW, STEPS, BATCH)

    zeros_big = jnp.zeros((NPAD, D), jnp.float32)

    deg = _deg_kernel(dst_sl)
    dcol = (deg[0, :N] + deg[1, :N])[:, None]

    b1r = b1.reshape(1, D)
    b2r = b2.reshape(1, D)

    hp1 = _tc_scale(_tc_mm(x, W1), dcol)
    acc1 = _msg_kernel(hp1, src_sl, dst_sl, zeros_big)
    hp2 = _tc_mid(acc1[0, :N], acc1[1, :N], hp1, dcol, W2, b1r)
    acc2 = _msg_kernel(hp2, src_sl, dst_sl, zeros_big)
    out = _tc_post(acc2[0, :N], acc2[1, :N], hp2, dcol, b2r)
    return out


# speedup vs baseline: 1.0502x; 1.0373x over previous
import functools

import jax
import jax.numpy as jnp
from jax import lax
from jax.experimental import pallas as pl
from jax.experimental.pallas import tpu as pltpu
from jax.experimental.pallas import tpu_sc as plsc

N = 10000
D = 128
E = 320000
NW = 32
NSUB = 16
BATCH = 128
STEPS = -(-E // (NW * BATCH))
E_PAD = NW * BATCH * STEPS
DUMP = 240
NPAD = N + DUMP
SHARE = NPAD // NSUB

_mesh = plsc.VectorSubcoreMesh(core_axis_name="c", subcore_axis_name="s")


@functools.partial(
    pl.kernel,
    out_type=jax.ShapeDtypeStruct((2, NPAD), jnp.float32),
    mesh=_mesh,
    compiler_params=pltpu.CompilerParams(needs_layout_passes=False),
    scratch_types=[
        pltpu.VMEM_SHARED((NSUB, NPAD), jnp.float32),
        pltpu.VMEM((STEPS, BATCH), jnp.int32),
        pltpu.VMEM((NPAD,), jnp.float32),
        pltpu.VMEM((NSUB, SHARE), jnp.float32),
        pltpu.VMEM((SHARE,), jnp.float32),
    ],
)
def _deg_kernel(dst_hbm, out_hbm, spm, slab, hist, mbuf, rbuf):
    c = lax.axis_index("c")
    s = lax.axis_index("s")
    w = c * NSUB + s
    pltpu.sync_copy(dst_hbm.at[w], slab)
    zvec = jnp.zeros((16,), jnp.float32)

    def zbody(i, carry):
        hist[pl.ds(i * 16, 16)] = zvec
        return carry

    lax.fori_loop(0, NPAD // 16, zbody, 0)

    def body(j, carry):
        for k in range(BATCH // 16):
            idx16 = slab[j, pl.ds(k * 16, 16)]
            cnt, last = plsc.scan_count(idx16)
            plsc.addupdate_scatter(hist, [idx16], cnt.astype(jnp.float32),
                                   mask=last)
        return carry

    lax.fori_loop(0, STEPS, body, 0)
    pltpu.sync_copy(hist, spm.at[s])
    plsc.subcore_barrier()
    pltpu.sync_copy(spm.at[:, pl.ds(s * SHARE, SHARE)], mbuf)

    def mbody(ci, carry):
        acc16 = zvec
        for r in range(NSUB):
            acc16 = acc16 + mbuf[r, pl.ds(ci * 16, 16)]
        rbuf[pl.ds(ci * 16, 16)] = acc16
        return carry

    lax.fori_loop(0, SHARE // 16, mbody, 0)
    pltpu.sync_copy(rbuf, out_hbm.at[c, pl.ds(s * SHARE, SHARE)])


@functools.partial(
    pl.kernel,
    out_type=jax.ShapeDtypeStruct((2, NPAD, D), jnp.float32),
    mesh=_mesh,
    scratch_types=[
        pltpu.VMEM_SHARED((NPAD, D), jnp.float32),
        pltpu.VMEM((STEPS, BATCH), jnp.int32),
        pltpu.VMEM((2, BATCH), jnp.int32),
        pltpu.VMEM((2, BATCH, D), jnp.float32),
        pltpu.SemaphoreType.DMA,
        pltpu.SemaphoreType.DMA,
    ],
)
def _msg_kernel(hp_hbm, src_hbm, dst_hbm, zeros_hbm, out_hbm,
                acc, srcv, dbuf, gbuf, semg, semi):
    c = lax.axis_index("c")
    s = lax.axis_index("s")
    w = c * NSUB + s
    pltpu.sync_copy(src_hbm.at[w], srcv)
    pltpu.async_copy(hp_hbm.at[srcv.at[0]], gbuf.at[0], semg)
    pltpu.async_copy(dst_hbm.at[w, 0], dbuf.at[0], semi)
    pltpu.sync_copy(zeros_hbm.at[pl.ds(s * SHARE, SHARE)],
                    acc.at[pl.ds(s * SHARE, SHARE)])
    plsc.subcore_barrier()

    def body(j, carry):
        b = lax.rem(j, 2)
        pltpu.make_async_copy(hp_hbm.at[srcv.at[j]], gbuf.at[b], semg).wait()
        pltpu.make_async_copy(dst_hbm.at[w, j], dbuf.at[b], semi).wait()

        @pl.when(j + 1 < STEPS)
        def _():
            pltpu.async_copy(hp_hbm.at[srcv.at[j + 1]], gbuf.at[1 - b], semg)
            pltpu.async_copy(dst_hbm.at[w, j + 1], dbuf.at[1 - b], semi)

        pltpu.sync_copy(gbuf.at[b], acc.at[dbuf.at[b]], add=True)
        return carry

    lax.fori_loop(0, STEPS, body, 0)
    plsc.subcore_barrier()
    pltpu.sync_copy(acc.at[pl.ds(s * SHARE, SHARE)],
                    out_hbm.at[c, pl.ds(s * SHARE, SHARE)])


_R = 1000


def _dinv_of(d_ref):
    return lax.rsqrt(d_ref[...] + 1.0)


def _tc_mm_body(x_ref, w_ref, o_ref):
    o_ref[...] = jnp.dot(x_ref[...], w_ref[...],
                         preferred_element_type=jnp.float32)


def _tc_scale_body(h_ref, d_ref, o_ref):
    o_ref[...] = _dinv_of(d_ref) * h_ref[...]


def _tc_mid_body(a0_ref, a1_ref, hp_ref, d_ref, w_ref, b_ref, o_ref):
    dinv = _dinv_of(d_ref)
    h = dinv * (a0_ref[0] + a1_ref[0] + hp_ref[...]) + b_ref[...]
    h = jnp.maximum(h, 0.0)
    o_ref[...] = dinv * jnp.dot(h, w_ref[...],
                                preferred_element_type=jnp.float32)


def _tc_post_body(a0_ref, a1_ref, hp_ref, d_ref, b_ref, o_ref):
    dinv = _dinv_of(d_ref)
    h = dinv * (a0_ref[0] + a1_ref[0] + hp_ref[...]) + b_ref[...]
    o_ref[...] = jnp.maximum(h, 0.0)


def _row_spec():
    return pl.BlockSpec((_R, D), lambda i: (i, 0))


def _deg_spec():
    return pl.BlockSpec((_R, 1), lambda i: (i, 0))


def _acc_spec(half):
    return pl.BlockSpec((1, _R, D), lambda i: (half, i, 0))


def _full_spec(shape):
    return pl.BlockSpec(shape, lambda i: tuple(0 for _ in shape))


def _tc_mm(x, w):
    return pl.pallas_call(
        _tc_mm_body,
        grid=(N // _R,),
        in_specs=[_row_spec(), _full_spec((D, D))],
        out_specs=_row_spec(),
        out_shape=jax.ShapeDtypeStruct((N, D), jnp.float32),
    )(x, w)


def _tc_scale(h, d):
    return pl.pallas_call(
        _tc_scale_body,
        grid=(N // _R,),
        in_specs=[_row_spec(), _deg_spec()],
        out_specs=_row_spec(),
        out_shape=jax.ShapeDtypeStruct((N, D), jnp.float32),
    )(h, d)


def _tc_mid(accs, hp, d, w, b):
    return pl.pallas_call(
        _tc_mid_body,
        grid=(N // _R,),
        in_specs=[_acc_spec(0), _acc_spec(1), _row_spec(), _deg_spec(),
                  _full_spec((D, D)), _full_spec((1, D))],
        out_specs=_row_spec(),
        out_shape=jax.ShapeDtypeStruct((N, D), jnp.float32),
    )(accs, accs, hp, d, w, b)


def _tc_post(accs, hp, d, b):
    return pl.pallas_call(
        _tc_post_body,
        grid=(N // _R,),
        in_specs=[_acc_spec(0), _acc_spec(1), _row_spec(), _deg_spec(),
                  _full_spec((1, D))],
        out_specs=_row_spec(),
        out_shape=jax.ShapeDtypeStruct((N, D), jnp.float32),
    )(accs, accs, hp, d, b)


def kernel(x, edge_index, W1, b1, W2, b2):
    src = edge_index[0].astype(jnp.int32)
    dst = edge_index[1].astype(jnp.int32)
    n_pad = E_PAD - E
    pad_idx = jnp.arange(n_pad, dtype=jnp.int32)
    pad_src = (pad_idx * 997) % N
    pad_dst = N + pad_idx % DUMP
    src_sl = jnp.concatenate([src, pad_src]).reshape(NW, STEPS, BATCH)
    dst_sl = jnp.concatenate([dst, pad_dst]).reshape(NW, STEPS, BATCH)

    zeros_big = jnp.zeros((NPAD, D), jnp.float32)

    deg = _deg_kernel(dst_sl)
    dcol = (deg[0, :N] + deg[1, :N])[:, None]

    b1r = b1.reshape(1, D)
    b2r = b2.reshape(1, D)

    hp1 = _tc_scale(_tc_mm(x, W1), dcol)
    acc1 = _msg_kernel(hp1, src_sl, dst_sl, zeros_big)
    hp2 = _tc_mid(acc1, hp1, dcol, W2, b1r)
    acc2 = _msg_kernel(hp2, src_sl, dst_sl, zeros_big)
    out = _tc_post(acc2, hp2, dcol, b2r)
    return out
